# baseline (speedup 1.0000x reference)
import dataclasses
import functools

import jax
import jax.numpy as jnp
from jax import lax
from jax.experimental import pallas as pl
from jax.experimental.pallas import tpu as pltpu
from jax.experimental.pallas import tpu_sc as plsc

LANES = 16
W = 16
NWORKERS = 32


def kernel(input_ids, content_table, pos_table, ln_gamma, ln_beta):
    B, L = input_ids.shape
    V, D = content_table.shape
    N = B * L
    nchunk = D // LANES
    rows_per_w = N // NWORKERS
    nwin = rows_per_w // W
    ids_flat = input_ids.reshape(N).astype(jnp.int32)

    mesh = plsc.VectorSubcoreMesh(
        core_axis_name="core", subcore_axis_name="subcore"
    )
    cp = pltpu.CompilerParams()
    if "needs_layout_passes" in pltpu.CompilerParams.__dataclass_fields__:
        cp = dataclasses.replace(cp, needs_layout_passes=False)

    @functools.partial(
        pl.kernel,
        out_type=jax.ShapeDtypeStruct((N, D), jnp.float32),
        mesh=mesh,
        compiler_params=cp,
        scratch_types=[
            pltpu.VMEM((rows_per_w,), jnp.int32),
            pltpu.VMEM((D,), jnp.float32),
            pltpu.VMEM((D,), jnp.float32),
            pltpu.VMEM((W, D), jnp.float32),
            pltpu.VMEM((W, D), jnp.float32),
            pltpu.VMEM((W, D), jnp.float32),
            pltpu.VMEM((W, D), jnp.float32),
            pltpu.VMEM((W, D), jnp.float32),
            pltpu.VMEM((W, D), jnp.float32),
            pltpu.VMEM((W, LANES), jnp.float32),
            pltpu.VMEM((W, LANES), jnp.float32),
            pltpu.SemaphoreType.DMA,
            pltpu.SemaphoreType.DMA,
            pltpu.SemaphoreType.DMA,
            pltpu.SemaphoreType.DMA,
            pltpu.SemaphoreType.DMA,
            pltpu.SemaphoreType.DMA,
        ],
    )
    def _emb_ln(content_hbm, ids_hbm, pos_hbm, g_hbm, b_hbm, out_hbm,
                idx_v, g_vmem, b_vmem, cont0, cont1, posb0, posb1,
                outb0, outb1, invs, m2s,
                gsem0, gsem1, psem0, psem1, osem0, osem1):
        cont = (cont0, cont1)
        posb = (posb0, posb1)
        outb = (outb0, outb1)
        gsem = (gsem0, gsem1)
        psem = (psem0, psem1)
        osem = (osem0, osem1)

        wid = lax.axis_index("core") * 16 + lax.axis_index("subcore")
        base = wid * rows_per_w
        pos_base = base % L

        pltpu.sync_copy(ids_hbm.at[pl.ds(base, rows_per_w)], idx_v)
        pltpu.sync_copy(g_hbm, g_vmem)
        pltpu.sync_copy(b_hbm, b_vmem)

        def start_in(k, b):
            pltpu.async_copy(
                content_hbm.at[idx_v.at[pl.ds(k * W, W)]], cont[b], gsem[b])
            pltpu.async_copy(
                pos_hbm.at[pl.ds(pos_base + k * W, W)], posb[b], psem[b])

        def wait_in(k, b):
            pltpu.make_async_copy(
                content_hbm.at[idx_v.at[pl.ds(k * W, W)]], cont[b],
                gsem[b]).wait()
            pltpu.make_async_copy(
                pos_hbm.at[pl.ds(pos_base + k * W, W)], posb[b],
                psem[b]).wait()

        def out_dma(k, b):
            return pltpu.make_async_copy(
                outb[b], out_hbm.at[pl.ds(base + k * W, W)], osem[b])

        start_in(0, 0)
        start_in(1, 1)

        @pl.loop(0, nwin, step=2)
        def _win2(k0):
            for bi in range(2):
                k = k0 + bi
                wait_in(k, bi)

                @pl.when(k >= 2)
                def _():
                    out_dma(k - 2, bi).wait()

                zero = jnp.zeros((LANES,), jnp.float32)
                HALF = W // 2
                ss = [None] * W
                qq = [None] * W
                for h in range(2):
                    carry0 = (tuple([zero] * HALF), tuple([zero] * HALF))

                    @plsc.parallel_loop(0, nchunk, carry=carry0)
                    def _p1(j, carry, h=h):
                        sh, qh = carry
                        sl = pl.ds(j * LANES, LANES)
                        ns = []
                        nq = []
                        for u in range(HALF):
                            t = h * HALF + u
                            e = cont[bi][t, sl] + posb[bi][t, sl]
                            outb[bi][t, sl] = e
                            ns.append(sh[u] + e)
                            nq.append(qh[u] + e * e)
                        return tuple(ns), tuple(nq)

                    sh, qh = _p1
                    for u in range(HALF):
                        ss[h * HALF + u] = sh[u]
                        qq[h * HALF + u] = qh[u]

                ys = []
                ms = []
                for t in range(W):
                    mean = jnp.sum(ss[t]) * (1.0 / D)
                    var = jnp.sum(qq[t]) * (1.0 / D) - mean * mean
                    vv = jnp.full((LANES,), var + 1e-5, jnp.float32)
                    bits = lax.bitcast_convert_type(vv, jnp.int32)
                    y = lax.bitcast_convert_type(
                        jnp.int32(0x5F3759DF) - (bits >> 1), jnp.float32)
                    y = y * (1.5 - 0.5 * vv * y * y)
                    y = y * (1.5 - 0.5 * vv * y * y)
                    ys.append(y)
                    ms.append(jnp.full((LANES,), mean, jnp.float32) * y)

                @pl.when(k + 2 < nwin)
                def _():
                    start_in(k + 2, bi)

                @plsc.parallel_loop(0, nchunk, unroll=4)
                def _p2(j):
                    sl = pl.ds(j * LANES, LANES)
                    g = g_vmem[sl]
                    b = b_vmem[sl]
                    for t in range(W):
                        e = outb[bi][t, sl]
                        n = e * ys[t] - ms[t]
                        outb[bi][t, sl] = n * g + b

                out_dma(k, bi).start()

        out_dma(nwin - 2, 0).wait()
        out_dma(nwin - 1, 1).wait()

    out = _emb_ln(content_table, ids_flat, pos_table, ln_gamma, ln_beta)
    return out.reshape(B, L, D)


# speedup vs baseline: 1.0145x; 1.0145x over previous
import dataclasses
import functools

import jax
import jax.numpy as jnp
from jax import lax
from jax.experimental import pallas as pl
from jax.experimental.pallas import tpu as pltpu
from jax.experimental.pallas import tpu_sc as plsc

LANES = 16
W = 16
NWORKERS = 32


def kernel(input_ids, content_table, pos_table, ln_gamma, ln_beta):
    B, L = input_ids.shape
    V, D = content_table.shape
    N = B * L
    nchunk = D // LANES
    rows_per_w = N // NWORKERS
    nwin = rows_per_w // W
    ids_flat = input_ids.reshape(N).astype(jnp.int32)

    mesh = plsc.VectorSubcoreMesh(
        core_axis_name="core", subcore_axis_name="subcore"
    )
    cp = pltpu.CompilerParams()
    if "needs_layout_passes" in pltpu.CompilerParams.__dataclass_fields__:
        cp = dataclasses.replace(cp, needs_layout_passes=False)

    @functools.partial(
        pl.kernel,
        out_type=jax.ShapeDtypeStruct((N, D), jnp.float32),
        mesh=mesh,
        compiler_params=cp,
        scratch_types=[
            pltpu.VMEM((rows_per_w,), jnp.int32),
            pltpu.VMEM((D,), jnp.float32),
            pltpu.VMEM((D,), jnp.float32),
            pltpu.VMEM((W, D), jnp.float32),
            pltpu.VMEM((W, D), jnp.float32),
            pltpu.VMEM((W, D), jnp.float32),
            pltpu.VMEM((W, D), jnp.float32),
            pltpu.VMEM((W, D), jnp.float32),
            pltpu.VMEM((W, D), jnp.float32),
            pltpu.SemaphoreType.DMA,
            pltpu.SemaphoreType.DMA,
            pltpu.SemaphoreType.DMA,
            pltpu.SemaphoreType.DMA,
            pltpu.SemaphoreType.DMA,
            pltpu.SemaphoreType.DMA,
        ],
    )
    def _emb_ln(content_hbm, ids_hbm, pos_hbm, g_hbm, b_hbm, out_hbm,
                idx_v, g_vmem, b_vmem, cont0, cont1, posb0, posb1,
                outb0, outb1,
                gsem0, gsem1, psem0, psem1, osem0, osem1):
        cont = (cont0, cont1)
        posb = (posb0, posb1)
        outb = (outb0, outb1)
        gsem = (gsem0, gsem1)
        psem = (psem0, psem1)
        osem = (osem0, osem1)

        wid = lax.axis_index("core") * 16 + lax.axis_index("subcore")
        base = wid * rows_per_w
        pos_base = base % L

        pltpu.sync_copy(ids_hbm.at[pl.ds(base, rows_per_w)], idx_v)
        pltpu.sync_copy(g_hbm, g_vmem)
        pltpu.sync_copy(b_hbm, b_vmem)

        def start_in(k, b):
            pltpu.async_copy(
                content_hbm.at[idx_v.at[pl.ds(k * W, W)]], cont[b], gsem[b])
            pltpu.async_copy(
                pos_hbm.at[pl.ds(pos_base + k * W, W)], posb[b], psem[b])

        def wait_in(k, b):
            pltpu.make_async_copy(
                content_hbm.at[idx_v.at[pl.ds(k * W, W)]], cont[b],
                gsem[b]).wait()
            pltpu.make_async_copy(
                pos_hbm.at[pl.ds(pos_base + k * W, W)], posb[b],
                psem[b]).wait()

        def out_dma(k, b):
            return pltpu.make_async_copy(
                outb[b], out_hbm.at[pl.ds(base + k * W, W)], osem[b])

        start_in(0, 0)
        start_in(1, 1)

        @pl.loop(0, nwin, step=2)
        def _win2(k0):
            for bi in range(2):
                k = k0 + bi
                wait_in(k, bi)

                @pl.when(k >= 2)
                def _():
                    out_dma(k - 2, bi).wait()

                zero = jnp.zeros((LANES,), jnp.float32)
                carry0 = (tuple([zero] * W), tuple([zero] * W))

                @plsc.parallel_loop(0, nchunk, carry=carry0)
                def _p1(j, carry):
                    ss, qq = carry
                    sl = pl.ds(j * LANES, LANES)
                    ns = []
                    nq = []
                    for t in range(W):
                        e = cont[bi][t, sl] + posb[bi][t, sl]
                        outb[bi][t, sl] = e
                        ns.append(ss[t] + e)
                        nq.append(qq[t] + e * e)
                    return tuple(ns), tuple(nq)

                ss, qq = _p1

                ys = []
                ms = []
                for t in range(W):
                    mean = jnp.sum(ss[t]) * (1.0 / D)
                    var = jnp.sum(qq[t]) * (1.0 / D) - mean * mean
                    vv = jnp.full((LANES,), var + 1e-5, jnp.float32)
                    bits = lax.bitcast_convert_type(vv, jnp.int32)
                    y = lax.bitcast_convert_type(
                        jnp.int32(0x5F3759DF) - (bits >> 1), jnp.float32)
                    y = y * (1.5 - 0.5 * vv * y * y)
                    y = y * (1.5 - 0.5 * vv * y * y)
                    ys.append(y)
                    ms.append(jnp.full((LANES,), mean, jnp.float32) * y)

                @pl.when(k + 2 < nwin)
                def _():
                    start_in(k + 2, bi)

                @plsc.parallel_loop(0, nchunk, unroll=4)
                def _p2(j):
                    sl = pl.ds(j * LANES, LANES)
                    g = g_vmem[sl]
                    b = b_vmem[sl]
                    for t in range(W):
                        e = outb[bi][t, sl]
                        n = e * ys[t] - ms[t]
                        outb[bi][t, sl] = n * g + b

                out_dma(k, bi).start()

        out_dma(nwin - 2, 0).wait()
        out_dma(nwin - 1, 1).wait()

    out = _emb_ln(content_table, ids_flat, pos_table, ln_gamma, ln_beta)
    return out.reshape(B, L, D)
